# Initial kernel scaffold; baseline (speedup 1.0000x reference)
import functools

import jax
import jax.numpy as jnp
from jax import lax
from jax.experimental import pallas as pl
from jax.experimental.pallas import tpu as pltpu
from jax.experimental.pallas import tpu_sc as plsc

B = 16384
K = 50
NUM_ATOMS = 1000000

NW = 32
R = B // NW
E = R * K
G = R // 16


def _scale_body(h_ref, rns_ref, out_ref):
    s = jnp.sum(h_ref[:])
    out_ref[:] = jnp.sqrt(jnp.maximum(rns_ref[:], 0.0)) / jnp.maximum(s, 1e-9)


def _tc_scale(rns, h):
    out = pl.pallas_call(
        _scale_body,
        out_shape=jax.ShapeDtypeStruct((128, 128), jnp.float32),
    )(h.reshape(1000, 1000), rns.reshape(128, 128))
    return out.reshape(B)


_mesh = plsc.VectorSubcoreMesh(core_axis_name="c", subcore_axis_name="s")


@functools.partial(
    pl.kernel,
    mesh=_mesh,
    out_type=jax.ShapeDtypeStruct((B,), jnp.float32),
    scratch_types=[
        pltpu.VMEM((E,), jnp.int32),
        pltpu.VMEM((E,), jnp.float32),
        pltpu.VMEM((E,), jnp.float32),
        pltpu.VMEM((R,), jnp.float32),
        pltpu.VMEM((R,), jnp.float32),
        pltpu.SemaphoreType.DMA,
        pltpu.SemaphoreType.DMA,
    ],
)
def _sc_gather_reduce(idx_hbm, vals_hbm, scale_hbm, table_hbm, out_hbm,
                      idx_v, h_v, vals_v, scale_v, out_v, sem_g, sem_l):
    wid = lax.axis_index("s") * 2 + lax.axis_index("c")
    base_e = wid * E
    base_r = wid * R

    pltpu.sync_copy(idx_hbm.at[pl.ds(base_e, E)], idx_v)
    gather = pltpu.async_copy(table_hbm.at[idx_v], h_v, sem_g)
    cp_vals = pltpu.async_copy(vals_hbm.at[pl.ds(base_e, E)], vals_v, sem_l)
    cp_scale = pltpu.async_copy(scale_hbm.at[pl.ds(base_r, R)], scale_v, sem_l)
    cp_vals.wait()
    cp_scale.wait()
    gather.wait()

    iota = lax.iota(jnp.int32, 16)
    iv0 = iota * K

    def body(g, _):
        iv = iv0 + g * (16 * K)
        acc = jnp.zeros((16,), jnp.float32)
        for _k in range(K):
            h16 = plsc.load_gather(h_v, [iv])
            v16 = plsc.load_gather(vals_v, [iv])
            acc = acc + jnp.maximum(v16, 0.0) * h16
            iv = iv + 1
        sc16 = scale_v[pl.ds(g * 16, 16)]
        out_v[pl.ds(g * 16, 16)] = acc * sc16
        return 0

    lax.fori_loop(0, G, body, 0)
    pltpu.sync_copy(out_v, out_hbm.at[pl.ds(base_r, R)])


def kernel(residual_norm_sq, topk_idx, topk_vals, atom_hazard_prior):
    idx = topk_idx.reshape(-1).astype(jnp.int32)
    vals = topk_vals.reshape(-1)
    scale = _tc_scale(residual_norm_sq, atom_hazard_prior)
    return _sc_gather_reduce(idx, vals, scale, atom_hazard_prior)


# speedup vs baseline: 1.3209x; 1.3209x over previous
import functools

import jax
import jax.numpy as jnp
from jax import lax
from jax.experimental import pallas as pl
from jax.experimental.pallas import tpu as pltpu
from jax.experimental.pallas import tpu_sc as plsc

B = 16384
K = 50
NUM_ATOMS = 1000000

NW = 32
R = B // NW
E = R * K
G = R // 16


def _scale_body(h_ref, rns_ref, out_ref):
    s = jnp.sum(h_ref[:])
    out_ref[:] = jnp.sqrt(jnp.maximum(rns_ref[:], 0.0)) / jnp.maximum(s, 1e-9)


def _tc_scale(rns, h):
    out = pl.pallas_call(
        _scale_body,
        out_shape=jax.ShapeDtypeStruct((128, 128), jnp.float32),
    )(h.reshape(1000, 1000), rns.reshape(128, 128))
    return out.reshape(B)


_mesh = plsc.VectorSubcoreMesh(core_axis_name="c", subcore_axis_name="s")


@functools.partial(
    pl.kernel,
    mesh=_mesh,
    out_type=jax.ShapeDtypeStruct((B,), jnp.float32),
    compiler_params=pltpu.CompilerParams(needs_layout_passes=False),
    scratch_types=[
        pltpu.VMEM((E,), jnp.int32),
        pltpu.VMEM((E,), jnp.float32),
        pltpu.VMEM((E,), jnp.float32),
        pltpu.VMEM((R,), jnp.float32),
        pltpu.VMEM((R,), jnp.float32),
        pltpu.SemaphoreType.DMA,
        pltpu.SemaphoreType.DMA,
    ],
)
def _sc_gather_reduce(idx_hbm, vals_hbm, scale_hbm, table_hbm, out_hbm,
                      idx_v, h_v, vals_v, scale_v, out_v, sem_g, sem_l):
    wid = lax.axis_index("s") * 2 + lax.axis_index("c")
    base_e = wid * E
    base_r = wid * R

    pltpu.sync_copy(idx_hbm.at[pl.ds(base_e, E)], idx_v)
    gather = pltpu.async_copy(table_hbm.at[idx_v], h_v, sem_g)
    cp_vals = pltpu.async_copy(vals_hbm.at[pl.ds(base_e, E)], vals_v, sem_l)
    cp_scale = pltpu.async_copy(scale_hbm.at[pl.ds(base_r, R)], scale_v, sem_l)
    cp_vals.wait()
    cp_scale.wait()
    gather.wait()

    iota = lax.iota(jnp.int32, 16)
    iv0 = iota * K

    def body(g, _):
        iv = iv0 + g * (16 * K)
        acc = jnp.zeros((16,), jnp.float32)
        for _k in range(K):
            h16 = plsc.load_gather(h_v, [iv])
            v16 = plsc.load_gather(vals_v, [iv])
            acc = acc + jnp.maximum(v16, 0.0) * h16
            iv = iv + 1
        sc16 = scale_v[pl.ds(g * 16, 16)]
        out_v[pl.ds(g * 16, 16)] = acc * sc16
        return 0

    lax.fori_loop(0, G, body, 0)
    pltpu.sync_copy(out_v, out_hbm.at[pl.ds(base_r, R)])


def kernel(residual_norm_sq, topk_idx, topk_vals, atom_hazard_prior):
    idx = topk_idx.reshape(-1).astype(jnp.int32)
    vals = topk_vals.reshape(-1)
    scale = _tc_scale(residual_norm_sq, atom_hazard_prior)
    return _sc_gather_reduce(idx, vals, scale, atom_hazard_prior)


# speedup vs baseline: 1.3745x; 1.0406x over previous
import functools

import jax
import jax.numpy as jnp
from jax import lax
from jax.experimental import pallas as pl
from jax.experimental.pallas import tpu as pltpu
from jax.experimental.pallas import tpu_sc as plsc

B = 16384
K = 50
NUM_ATOMS = 1000000

NW = 32
R = B // NW
E = R * K
G = R // 16


def _finish_body(h_ref, rns_ref, rowsum_ref, out_ref):
    s = jnp.sum(h_ref[:])
    novelty = jnp.sqrt(jnp.maximum(rns_ref[:], 0.0))
    out_ref[:] = rowsum_ref[:] * novelty / jnp.maximum(s, 1e-9)


def _tc_finish(h, rns, rowsum):
    out = pl.pallas_call(
        _finish_body,
        out_shape=jax.ShapeDtypeStruct((128, 128), jnp.float32),
    )(h.reshape(1000, 1000), rns.reshape(128, 128), rowsum.reshape(128, 128))
    return out.reshape(B)


_mesh = plsc.VectorSubcoreMesh(core_axis_name="c", subcore_axis_name="s")


@functools.partial(
    pl.kernel,
    mesh=_mesh,
    out_type=jax.ShapeDtypeStruct((B,), jnp.float32),
    compiler_params=pltpu.CompilerParams(needs_layout_passes=False),
    scratch_types=[
        pltpu.VMEM((E,), jnp.int32),
        pltpu.VMEM((E,), jnp.float32),
        pltpu.VMEM((E,), jnp.float32),
        pltpu.VMEM((R,), jnp.float32),
        pltpu.SemaphoreType.DMA,
        pltpu.SemaphoreType.DMA,
    ],
)
def _sc_gather_reduce(idx_hbm, vals_hbm, table_hbm, out_hbm,
                      idx_v, h_v, vals_v, out_v, sem_g, sem_l):
    wid = lax.axis_index("s") * 2 + lax.axis_index("c")
    base_e = wid * E
    base_r = wid * R

    pltpu.sync_copy(idx_hbm.at[pl.ds(base_e, E)], idx_v)
    gather = pltpu.async_copy(table_hbm.at[idx_v], h_v, sem_g)
    cp_vals = pltpu.async_copy(vals_hbm.at[pl.ds(base_e, E)], vals_v, sem_l)
    cp_vals.wait()
    gather.wait()

    iota = lax.iota(jnp.int32, 16)
    iv0 = iota * K

    def body(g, _):
        iv = iv0 + g * (16 * K)
        acc = jnp.zeros((16,), jnp.float32)
        for _k in range(K):
            h16 = plsc.load_gather(h_v, [iv])
            v16 = plsc.load_gather(vals_v, [iv])
            acc = acc + jnp.maximum(v16, 0.0) * h16
            iv = iv + 1
        out_v[pl.ds(g * 16, 16)] = acc
        return 0

    lax.fori_loop(0, G, body, 0)
    pltpu.sync_copy(out_v, out_hbm.at[pl.ds(base_r, R)])


def kernel(residual_norm_sq, topk_idx, topk_vals, atom_hazard_prior):
    idx = topk_idx.reshape(-1).astype(jnp.int32)
    vals = topk_vals.reshape(-1)
    rowsum = _sc_gather_reduce(idx, vals, atom_hazard_prior)
    return _tc_finish(atom_hazard_prior, residual_norm_sq, rowsum)


# speedup vs baseline: 1.4793x; 1.0762x over previous
import functools

import numpy as np
import jax
import jax.numpy as jnp
from jax import lax
from jax.experimental import pallas as pl
from jax.experimental.pallas import tpu as pltpu
from jax.experimental.pallas import tpu_sc as plsc

B = 16384
K = 50
NUM_ATOMS = 1000000

NW = 32
R = B // NW
H = R // 2
E = R * K
EH = H * K
NBLK = H // 8
NVEC = 8 * K // 16


def _finish_body(h_ref, rns_ref, rowsum_ref, out_ref):
    s = jnp.sum(h_ref[:])
    novelty = jnp.sqrt(jnp.maximum(rns_ref[:], 0.0))
    out_ref[:] = rowsum_ref[:] * novelty / jnp.maximum(s, 1e-9)


def _tc_finish(h, rns, rowsum):
    out = pl.pallas_call(
        _finish_body,
        out_shape=jax.ShapeDtypeStruct((128, 128), jnp.float32),
    )(h.reshape(1000, 1000), rns.reshape(128, 128), rowsum.reshape(128, 128))
    return out.reshape(B)


_mesh = plsc.VectorSubcoreMesh(core_axis_name="c", subcore_axis_name="s")


@functools.partial(
    pl.kernel,
    mesh=_mesh,
    out_type=jax.ShapeDtypeStruct((B,), jnp.float32),
    compiler_params=pltpu.CompilerParams(needs_layout_passes=False),
    scratch_types=[
        pltpu.VMEM((H, K), jnp.int32),
        pltpu.VMEM((H, K), jnp.float32),
        pltpu.VMEM((E,), jnp.int32),
        pltpu.VMEM((E,), jnp.float32),
        pltpu.VMEM((R,), jnp.float32),
        pltpu.SemaphoreType.DMA,
        pltpu.SemaphoreType.DMA,
    ],
)
def _sc_gather_reduce(idx_hbm, vals_hbm, table_hbm, out_hbm,
                      idx2d, vals2d, idxf, hf, out_v, sem_g, sem_l):
    wid = lax.axis_index("s") * 2 + lax.axis_index("c")
    base_r = wid * R

    iota16 = lax.iota(jnp.int32, 16)
    rows_c = [(iota16 + 16 * i) // K for i in range(NVEC)]
    cols_c = [(iota16 + 16 * i) % K for i in range(NVEC)]

    def pack_half(hh):
        def blk_body(blk, _):
            fbase = hh * EH + blk * (8 * K)
            for i in range(NVEC):
                r = rows_c[i] + blk * 8
                v = plsc.load_gather(idx2d, [r, cols_c[i]])
                idxf[pl.ds(fbase + 16 * i, 16)] = v
            return 0
        lax.fori_loop(0, NBLK, blk_body, 0)

    def compute_half(hh):
        def g_body(g, _):
            rows = lax.iota(jnp.int32, 16) + g * 16
            fbase = hh * EH + g * 16 * K
            acc = jnp.zeros((16,), jnp.float32)
            for k in range(K):
                iv = lax.iota(jnp.int32, 16) * K + (fbase + k)
                h16 = plsc.load_gather(hf, [iv])
                v16 = plsc.load_gather(
                    vals2d, [rows, jnp.full((16,), k, jnp.int32)]
                )
                acc = acc + jnp.maximum(v16, 0.0) * h16
            out_v[pl.ds(hh * H + g * 16, 16)] = acc
            return 0
        lax.fori_loop(0, H // 16, g_body, 0)

    pltpu.sync_copy(idx_hbm.at[pl.ds(base_r, H), :], idx2d)
    pack_half(0)
    g0 = pltpu.async_copy(table_hbm.at[idxf.at[pl.ds(0, EH)]],
                          hf.at[pl.ds(0, EH)], sem_g)
    pltpu.sync_copy(idx_hbm.at[pl.ds(base_r + H, H), :], idx2d)
    pack_half(1)
    g1 = pltpu.async_copy(table_hbm.at[idxf.at[pl.ds(EH, EH)]],
                          hf.at[pl.ds(EH, EH)], sem_g)
    pltpu.sync_copy(vals_hbm.at[pl.ds(base_r, H), :], vals2d)
    g0.wait()
    compute_half(0)
    pltpu.sync_copy(vals_hbm.at[pl.ds(base_r + H, H), :], vals2d)
    g1.wait()
    compute_half(1)

    pltpu.sync_copy(out_v, out_hbm.at[pl.ds(base_r, R)])


def kernel(residual_norm_sq, topk_idx, topk_vals, atom_hazard_prior):
    idx = topk_idx.astype(jnp.int32)
    rowsum = _sc_gather_reduce(idx, topk_vals, atom_hazard_prior)
    return _tc_finish(atom_hazard_prior, residual_norm_sq, rowsum)
